# Initial kernel scaffold; baseline (speedup 1.0000x reference)
import jax
import jax.numpy as jnp
from jax.experimental import pallas as pl


def kernel(atomic_numbers, pos, edge_index, id3_ba, id3_ca, batch, atom_table, W_edge, W_rbf3, W_cbf3, W_rbf_h, W_rbf_out, Wb_db, Wb_rbf3p, Wb_down, Wb_cbfp, Wb_bil, Wb_up, Wb_res1, Wb_res2, Wb_res3, Wb_rbf_hp, Wb_atom, Wb_concat, Wo_rbfp, Wo_atom, Wo_final):
    raise NotImplementedError("write your pallas kernel here")


# speedup vs baseline: 2.0216x; 2.0216x over previous
import functools

import jax
import jax.numpy as jnp
import numpy as np
from jax.experimental import pallas as pl
from jax.experimental.pallas import tpu as pltpu

INV2 = 1.0 / np.sqrt(2.0)
CUT = 6.0
NS = 7
NGRAPH = 32


def _swish(x):
    return x * jax.nn.sigmoid(x)


def _ln(x):
    mu = jnp.mean(x, axis=-1, keepdims=True)
    v = jnp.mean((x - mu) * (x - mu), axis=-1, keepdims=True)
    return (x - mu) * jax.lax.rsqrt(v + 1e-5)


def _row_spec(blk, w):
    return pl.BlockSpec((blk, w), lambda i: (i, 0))


def _full_spec(shape):
    nd = len(shape)
    return pl.BlockSpec(shape, lambda i: (0,) * nd)


def _rowcall(body, nrows, blk, row_ins, full_ins, out_widths):
    grid = (nrows // blk,)
    in_specs = [_row_spec(blk, a.shape[1]) for a in row_ins]
    in_specs += [_full_spec(a.shape) for a in full_ins]
    out_specs = [_row_spec(blk, w) for w in out_widths]
    out_shape = [jax.ShapeDtypeStruct((nrows, w), jnp.float32) for w in out_widths]
    if len(out_widths) == 1:
        out_specs = out_specs[0]
        out_shape = out_shape[0]
    return pl.pallas_call(
        body,
        grid=grid,
        in_specs=in_specs,
        out_specs=out_specs,
        out_shape=out_shape,
        compiler_params=pltpu.CompilerParams(
            dimension_semantics=("arbitrary",)),
    )(*row_ins, *full_ins)


def _k1_body(nr, d_ref, wr3_ref, wrh_ref, wro_ref, wcbf_ref,
             rbf_ref, r3_ref, rh_ref, ro_ref, renv_ref):
    d = d_ref[...]
    d5 = d * d * d * d * d
    env = 1.0 - 21.0 * d5 + 35.0 * d5 * d - 15.0 * d5 * d * d
    env = jnp.where(d < 1.0, env, 0.0)
    step = 1.0 / (nr - 1)
    offs = jax.lax.broadcasted_iota(
        jnp.int32, (1, nr), 1).astype(jnp.float32) * step
    coeff = -0.5 / (step * step)
    delta = d - offs
    rbf = jnp.exp(coeff * delta * delta) * env
    rbf_ref[...] = rbf
    r3_ref[...] = jnp.dot(rbf, wr3_ref[...], preferred_element_type=jnp.float32)
    rh_ref[...] = jnp.dot(rbf, wrh_ref[...], preferred_element_type=jnp.float32)
    ro_ref[...] = jnp.dot(rbf, wro_ref[...], preferred_element_type=jnp.float32)
    renv_ref[...] = jnp.dot(rbf, wcbf_ref[...], preferred_element_type=jnp.float32)


def _concat_body(res_w, a_ref, b_ref, c_ref, w1_ref, w2_ref, w3_ref, *rest):
    out_ref = rest[-1]
    x = jnp.dot(a_ref[...], w1_ref[...], preferred_element_type=jnp.float32)
    x = x + jnp.dot(b_ref[...], w2_ref[...], preferred_element_type=jnp.float32)
    x = x + jnp.dot(c_ref[...], w3_ref[...], preferred_element_type=jnp.float32)
    x = _swish(x)
    for j in range(res_w):
        wa = rest[2 * j][...]
        wb = rest[2 * j + 1][...]
        t = _swish(jnp.dot(x, wa, preferred_element_type=jnp.float32))
        t = _swish(jnp.dot(t, wb, preferred_element_type=jnp.float32))
        x = (x + t) * INV2
    out_ref[...] = x


def _k3_body(m_ref, r3_ref, wdb_ref, wr3p_ref, wdown_ref, xt_ref):
    xb = _swish(jnp.dot(m_ref[...], wdb_ref[...], preferred_element_type=jnp.float32))
    xb = xb * jnp.dot(r3_ref[...], wr3p_ref[...], preferred_element_type=jnp.float32)
    xt_ref[...] = _swish(jnp.dot(xb, wdown_ref[...], preferred_element_type=jnp.float32))


def _k4_body(cbf_ref, x3g_ref, wcbfp_ref, x3_ref):
    x3_ref[...] = x3g_ref[...] * jnp.dot(
        cbf_ref[...], wcbfp_ref[...], preferred_element_type=jnp.float32)


def _k5_body(m_ref, xe_ref, rh_ref, wbil_ref, wup_ref, wres1_ref, wres2_ref,
             whp_ref, mnew_ref, mscaled_ref):
    xe = _swish(jnp.dot(xe_ref[...], wbil_ref[...], preferred_element_type=jnp.float32))
    xe = _swish(jnp.dot(xe, wup_ref[...], preferred_element_type=jnp.float32))
    x = (m_ref[...] + xe) * INV2
    w1 = wres1_ref[...]
    t = _swish(jnp.dot(x, w1[0], preferred_element_type=jnp.float32))
    t = _swish(jnp.dot(t, w1[1], preferred_element_type=jnp.float32))
    x = (x + t) * INV2
    w2 = wres2_ref[...]
    for j in range(2):
        t = _swish(jnp.dot(x, w2[j, 0], preferred_element_type=jnp.float32))
        t = _swish(jnp.dot(t, w2[j, 1], preferred_element_type=jnp.float32))
        x = (x + t) * INV2
    mnew_ref[...] = x
    mscaled_ref[...] = x * jnp.dot(
        rh_ref[...], whp_ref[...], preferred_element_type=jnp.float32)


def _k6a_body(xa_ref, h_ref, watom_ref, hnew_ref):
    x = xa_ref[...]
    w = watom_ref[...]
    for l in range(w.shape[0]):
        x = (x + _swish(jnp.dot(x, w[l], preferred_element_type=jnp.float32))) * INV2
    hnew_ref[...] = (h_ref[...] + x) * INV2


def _k6b_body(xa_ref, watom_ref, wfin_ref, e_ref):
    x = xa_ref[...]
    w = watom_ref[...]
    for l in range(w.shape[0]):
        x = (x + _swish(jnp.dot(x, w[l], preferred_element_type=jnp.float32))) * INV2
    x = _ln(x)
    e_ref[...] = jnp.dot(x, wfin_ref[...], preferred_element_type=jnp.float32)


def _k8_body(m_ref, ro_ref, worbfp_ref, x_ref):
    x_ref[...] = _ln(m_ref[...]) * jnp.dot(
        ro_ref[...], worbfp_ref[...], preferred_element_type=jnp.float32)


def _pick_blk(n, want):
    b = min(want, n)
    while n % b:
        b -= 1
    return b


def kernel(atomic_numbers, pos, edge_index, id3_ba, id3_ca, batch, atom_table,
           W_edge, W_rbf3, W_cbf3, W_rbf_h, W_rbf_out, Wb_db, Wb_rbf3p,
           Wb_down, Wb_cbfp, Wb_bil, Wb_up, Wb_res1, Wb_res2, Wb_res3,
           Wb_rbf_hp, Wb_atom, Wb_concat, Wo_rbfp, Wo_atom, Wo_final):
    N = pos.shape[0]
    E = edge_index.shape[1]
    T = id3_ba.shape[0]
    NR = W_rbf3.shape[0]
    EA = atom_table.shape[1]
    EE = W_edge.shape[1]
    NB = Wb_db.shape[0]
    NG = NGRAPH

    blk_e = _pick_blk(E, 640)
    blk_t = _pick_blk(T, 640)
    blk_n = _pick_blk(N, 1000)

    idx_s = edge_index[0]
    idx_t = edge_index[1]

    vec = pos[idx_t] - pos[idx_s]
    D = jnp.sqrt(jnp.sum(vec * vec, axis=-1) + 1e-12)
    V = vec / D[:, None]
    d = (D / CUT)[:, None]

    wcbf = jnp.transpose(W_cbf3, (1, 0, 2)).reshape(NR, NS * W_cbf3.shape[2])
    rbf, rbf3, rbf_hp, rbf_outp, rbf_env = _rowcall(
        functools.partial(_k1_body, NR), E, blk_e,
        [d], [W_rbf3, W_rbf_h, W_rbf_out, wcbf],
        [NR, W_rbf3.shape[1], W_rbf_h.shape[1], W_rbf_out.shape[1], wcbf.shape[1]])

    cosang = jnp.clip(jnp.sum(V[id3_ba] * V[id3_ca], axis=-1), -1.0, 1.0)
    sph_list = [jnp.ones_like(cosang), cosang]
    for l in range(2, NS):
        sph_list.append(
            ((2 * l - 1) * cosang * sph_list[l - 1] - (l - 1) * sph_list[l - 2]) / l)
    sph = jnp.stack(sph_list, axis=1)
    ECBF = W_cbf3.shape[2]
    cbf_t = jnp.sum(
        rbf_env[id3_ca].reshape(T, NS, ECBF) * sph[:, :, None], axis=1)

    h = atom_table[atomic_numbers]

    w1, w2, w3 = W_edge[:EA], W_edge[EA:2 * EA], W_edge[2 * EA:]
    m = _rowcall(functools.partial(_concat_body, 0), E, blk_e,
                 [h[idx_s], h[idx_t], rbf], [w1, w2, w3], [EE])

    def outblock(i, h_in, m_in):
        x = _rowcall(_k8_body, E, blk_e, [m_in, rbf_outp], [Wo_rbfp[i]], [EE])
        xa = jax.ops.segment_sum(x, idx_t, num_segments=N)
        return _rowcall(_k6b_body, N, blk_n, [xa], [Wo_atom[i], Wo_final[i]], [1])

    Eacc = outblock(0, h, m)
    for i in range(NB):
        xt = _rowcall(_k3_body, E, blk_e, [m, rbf3],
                      [Wb_db[i], Wb_rbf3p[i], Wb_down[i]], [Wb_down.shape[2]])
        x3 = _rowcall(_k4_body, T, blk_t, [cbf_t, xt[id3_ba]], [Wb_cbfp[i]],
                      [Wb_cbfp.shape[2]])
        xe = jax.ops.segment_sum(x3, id3_ca, num_segments=E)
        mnew, mscaled = _rowcall(
            _k5_body, E, blk_e, [m, xe, rbf_hp],
            [Wb_bil[i], Wb_up[i], Wb_res1[i, 0], Wb_res2[i], Wb_rbf_hp[i]],
            [EE, EE])
        xa = jax.ops.segment_sum(mscaled, idx_t, num_segments=N)
        h = _rowcall(_k6a_body, N, blk_n, [xa, h], [Wb_atom[i]], [EA])
        m = _rowcall(
            functools.partial(_concat_body, 1), E, blk_e,
            [h[idx_s], h[idx_t], mnew],
            [Wb_concat[i, :EA], Wb_concat[i, EA:2 * EA], Wb_concat[i, 2 * EA:],
             Wb_res3[i, 0, 0], Wb_res3[i, 0, 1]], [EE])
        Eacc = Eacc + outblock(i + 1, h, m)

    return jax.ops.segment_sum(Eacc[:, 0], batch, num_segments=NG)


# speedup vs baseline: 3.1159x; 1.5413x over previous
import functools

import jax
import jax.numpy as jnp
import numpy as np
from jax import lax
from jax.experimental import pallas as pl
from jax.experimental.pallas import tpu as pltpu
from jax.experimental.pallas import tpu_sc as plsc

_SC_CORES = 2
_SC_SUBCORES = 16
_SC_WORKERS = _SC_CORES * _SC_SUBCORES


def _sc_chunking(bw):
    ch = 0
    for c in range(128, 7, -1):
        if c % 8 == 0 and bw % c == 0:
            ch = c
            break
    nch = bw // ch
    fire = 1
    for k in (8, 5, 4, 2):
        if nch % k == 0:
            fire = k
            break
    return ch, nch, fire


def _sc_gather(table, idx):
    r, d = table.shape
    b = idx.shape[0]
    bw = b // _SC_WORKERS
    ch, nch, fire = _sc_chunking(bw)
    ng = nch // fire
    grp = ch * fire
    idx2 = idx.reshape(_SC_WORKERS, nch, ch)

    mesh = plsc.VectorSubcoreMesh(core_axis_name="c", subcore_axis_name="s")

    @functools.partial(
        pl.kernel,
        out_type=jax.ShapeDtypeStruct((b, d), jnp.float32),
        mesh=mesh,
        scratch_types=[
            pltpu.VMEM((nch, ch), jnp.int32),
            pltpu.VMEM((grp, d), jnp.float32),
            pltpu.SemaphoreType.DMA,
        ],
        compiler_params=pltpu.CompilerParams(use_tc_tiling_on_sc=False),
    )
    def gather_kernel(table_hbm, idx_hbm, out_hbm, idx_v, rows_v, sem):
        wid = lax.axis_index("s") * _SC_CORES + lax.axis_index("c")
        base = wid * bw
        pltpu.sync_copy(idx_hbm.at[wid], idx_v)

        def group(g, _):
            cps = []
            for bb in range(fire):
                cps.append(pltpu.async_copy(
                    table_hbm.at[idx_v.at[g * fire + bb]],
                    rows_v.at[pl.ds(bb * ch, ch)], sem))
            for cp in cps:
                cp.wait()
            pltpu.sync_copy(rows_v, out_hbm.at[pl.ds(base + g * grp, grp)])
            return 0

        lax.fori_loop(0, ng, group, 0)

    return gather_kernel(table, idx2)

INV2 = 1.0 / np.sqrt(2.0)
CUT = 6.0
NS = 7
NGRAPH = 32


def _swish(x):
    return x * jax.nn.sigmoid(x)


def _ln(x):
    mu = jnp.mean(x, axis=-1, keepdims=True)
    v = jnp.mean((x - mu) * (x - mu), axis=-1, keepdims=True)
    return (x - mu) * jax.lax.rsqrt(v + 1e-5)


def _row_spec(blk, w):
    return pl.BlockSpec((blk, w), lambda i: (i, 0))


def _full_spec(shape):
    nd = len(shape)
    return pl.BlockSpec(shape, lambda i: (0,) * nd)


def _rowcall(body, nrows, blk, row_ins, full_ins, out_widths):
    grid = (nrows // blk,)
    in_specs = [_row_spec(blk, a.shape[1]) for a in row_ins]
    in_specs += [_full_spec(a.shape) for a in full_ins]
    out_specs = [_row_spec(blk, w) for w in out_widths]
    out_shape = [jax.ShapeDtypeStruct((nrows, w), jnp.float32) for w in out_widths]
    if len(out_widths) == 1:
        out_specs = out_specs[0]
        out_shape = out_shape[0]
    return pl.pallas_call(
        body,
        grid=grid,
        in_specs=in_specs,
        out_specs=out_specs,
        out_shape=out_shape,
        compiler_params=pltpu.CompilerParams(
            dimension_semantics=("arbitrary",)),
    )(*row_ins, *full_ins)


def _k1_body(nr, d_ref, wr3_ref, wrh_ref, wro_ref, wcbf_ref,
             rbf_ref, r3_ref, rh_ref, ro_ref, renv_ref):
    d = d_ref[...]
    d5 = d * d * d * d * d
    env = 1.0 - 21.0 * d5 + 35.0 * d5 * d - 15.0 * d5 * d * d
    env = jnp.where(d < 1.0, env, 0.0)
    step = 1.0 / (nr - 1)
    offs = jax.lax.broadcasted_iota(
        jnp.int32, (1, nr), 1).astype(jnp.float32) * step
    coeff = -0.5 / (step * step)
    delta = d - offs
    rbf = jnp.exp(coeff * delta * delta) * env
    rbf_ref[...] = rbf
    r3_ref[...] = jnp.dot(rbf, wr3_ref[...], preferred_element_type=jnp.float32)
    rh_ref[...] = jnp.dot(rbf, wrh_ref[...], preferred_element_type=jnp.float32)
    ro_ref[...] = jnp.dot(rbf, wro_ref[...], preferred_element_type=jnp.float32)
    renv_ref[...] = jnp.dot(rbf, wcbf_ref[...], preferred_element_type=jnp.float32)


def _concat_body(res_w, a_ref, b_ref, c_ref, w1_ref, w2_ref, w3_ref, *rest):
    out_ref = rest[-1]
    x = jnp.dot(a_ref[...], w1_ref[...], preferred_element_type=jnp.float32)
    x = x + jnp.dot(b_ref[...], w2_ref[...], preferred_element_type=jnp.float32)
    x = x + jnp.dot(c_ref[...], w3_ref[...], preferred_element_type=jnp.float32)
    x = _swish(x)
    for j in range(res_w):
        wa = rest[2 * j][...]
        wb = rest[2 * j + 1][...]
        t = _swish(jnp.dot(x, wa, preferred_element_type=jnp.float32))
        t = _swish(jnp.dot(t, wb, preferred_element_type=jnp.float32))
        x = (x + t) * INV2
    out_ref[...] = x


def _k3_body(m_ref, r3_ref, wdb_ref, wr3p_ref, wdown_ref, xt_ref):
    xb = _swish(jnp.dot(m_ref[...], wdb_ref[...], preferred_element_type=jnp.float32))
    xb = xb * jnp.dot(r3_ref[...], wr3p_ref[...], preferred_element_type=jnp.float32)
    xt_ref[...] = _swish(jnp.dot(xb, wdown_ref[...], preferred_element_type=jnp.float32))


def _k4_body(cbf_ref, x3g_ref, wcbfp_ref, x3_ref):
    x3_ref[...] = x3g_ref[...] * jnp.dot(
        cbf_ref[...], wcbfp_ref[...], preferred_element_type=jnp.float32)


def _k5_body(m_ref, xe_ref, rh_ref, wbil_ref, wup_ref, wres1_ref, wres2_ref,
             whp_ref, mnew_ref, mscaled_ref):
    xe = _swish(jnp.dot(xe_ref[...], wbil_ref[...], preferred_element_type=jnp.float32))
    xe = _swish(jnp.dot(xe, wup_ref[...], preferred_element_type=jnp.float32))
    x = (m_ref[...] + xe) * INV2
    w1 = wres1_ref[...]
    t = _swish(jnp.dot(x, w1[0], preferred_element_type=jnp.float32))
    t = _swish(jnp.dot(t, w1[1], preferred_element_type=jnp.float32))
    x = (x + t) * INV2
    w2 = wres2_ref[...]
    for j in range(2):
        t = _swish(jnp.dot(x, w2[j, 0], preferred_element_type=jnp.float32))
        t = _swish(jnp.dot(t, w2[j, 1], preferred_element_type=jnp.float32))
        x = (x + t) * INV2
    mnew_ref[...] = x
    mscaled_ref[...] = x * jnp.dot(
        rh_ref[...], whp_ref[...], preferred_element_type=jnp.float32)


def _k6a_body(xa_ref, h_ref, watom_ref, hnew_ref):
    x = xa_ref[...]
    w = watom_ref[...]
    for l in range(w.shape[0]):
        x = (x + _swish(jnp.dot(x, w[l], preferred_element_type=jnp.float32))) * INV2
    hnew_ref[...] = (h_ref[...] + x) * INV2


def _k6b_body(xa_ref, watom_ref, wfin_ref, e_ref):
    x = xa_ref[...]
    w = watom_ref[...]
    for l in range(w.shape[0]):
        x = (x + _swish(jnp.dot(x, w[l], preferred_element_type=jnp.float32))) * INV2
    x = _ln(x)
    e_ref[...] = jnp.dot(x, wfin_ref[...], preferred_element_type=jnp.float32)


def _k8_body(m_ref, ro_ref, worbfp_ref, x_ref):
    x_ref[...] = _ln(m_ref[...]) * jnp.dot(
        ro_ref[...], worbfp_ref[...], preferred_element_type=jnp.float32)


def _pick_blk(n, want):
    b = min(want, n)
    while n % b:
        b -= 1
    return b


def kernel(atomic_numbers, pos, edge_index, id3_ba, id3_ca, batch, atom_table,
           W_edge, W_rbf3, W_cbf3, W_rbf_h, W_rbf_out, Wb_db, Wb_rbf3p,
           Wb_down, Wb_cbfp, Wb_bil, Wb_up, Wb_res1, Wb_res2, Wb_res3,
           Wb_rbf_hp, Wb_atom, Wb_concat, Wo_rbfp, Wo_atom, Wo_final):
    N = pos.shape[0]
    E = edge_index.shape[1]
    T = id3_ba.shape[0]
    NR = W_rbf3.shape[0]
    EA = atom_table.shape[1]
    EE = W_edge.shape[1]
    NB = Wb_db.shape[0]
    NG = NGRAPH

    blk_e = _pick_blk(E, 640)
    blk_t = _pick_blk(T, 640)
    blk_n = _pick_blk(N, 1000)

    idx_s = edge_index[0]
    idx_t = edge_index[1]

    vec = pos[idx_t] - pos[idx_s]
    D = jnp.sqrt(jnp.sum(vec * vec, axis=-1) + 1e-12)
    V = vec / D[:, None]
    d = (D / CUT)[:, None]

    wcbf = jnp.transpose(W_cbf3, (1, 0, 2)).reshape(NR, NS * W_cbf3.shape[2])
    rbf, rbf3, rbf_hp, rbf_outp, rbf_env = _rowcall(
        functools.partial(_k1_body, NR), E, blk_e,
        [d], [W_rbf3, W_rbf_h, W_rbf_out, wcbf],
        [NR, W_rbf3.shape[1], W_rbf_h.shape[1], W_rbf_out.shape[1], wcbf.shape[1]])

    Vp = jnp.concatenate([V, jnp.zeros((E, 13), jnp.float32)], axis=1)
    Vba = _sc_gather(Vp, id3_ba)
    Vca = _sc_gather(Vp, id3_ca)
    cosang = jnp.clip(jnp.sum(Vba * Vca, axis=-1), -1.0, 1.0)
    sph_list = [jnp.ones_like(cosang), cosang]
    for l in range(2, NS):
        sph_list.append(
            ((2 * l - 1) * cosang * sph_list[l - 1] - (l - 1) * sph_list[l - 2]) / l)
    sph = jnp.stack(sph_list, axis=1)
    ECBF = W_cbf3.shape[2]
    renvg = _sc_gather(rbf_env, id3_ca)
    cbf_t = jnp.sum(renvg.reshape(T, NS, ECBF) * sph[:, :, None], axis=1)

    h = atom_table[atomic_numbers]

    w1, w2, w3 = W_edge[:EA], W_edge[EA:2 * EA], W_edge[2 * EA:]
    m = _rowcall(functools.partial(_concat_body, 0), E, blk_e,
                 [_sc_gather(h, idx_s), _sc_gather(h, idx_t), rbf],
                 [w1, w2, w3], [EE])

    def outblock(i, h_in, m_in):
        x = _rowcall(_k8_body, E, blk_e, [m_in, rbf_outp], [Wo_rbfp[i]], [EE])
        xa = jax.ops.segment_sum(x, idx_t, num_segments=N)
        return _rowcall(_k6b_body, N, blk_n, [xa], [Wo_atom[i], Wo_final[i]], [1])

    Eacc = outblock(0, h, m)
    for i in range(NB):
        xt = _rowcall(_k3_body, E, blk_e, [m, rbf3],
                      [Wb_db[i], Wb_rbf3p[i], Wb_down[i]], [Wb_down.shape[2]])
        x3 = _rowcall(_k4_body, T, blk_t, [cbf_t, _sc_gather(xt, id3_ba)],
                      [Wb_cbfp[i]], [Wb_cbfp.shape[2]])
        xe = jax.ops.segment_sum(x3, id3_ca, num_segments=E)
        mnew, mscaled = _rowcall(
            _k5_body, E, blk_e, [m, xe, rbf_hp],
            [Wb_bil[i], Wb_up[i], Wb_res1[i, 0], Wb_res2[i], Wb_rbf_hp[i]],
            [EE, EE])
        xa = jax.ops.segment_sum(mscaled, idx_t, num_segments=N)
        h = _rowcall(_k6a_body, N, blk_n, [xa, h], [Wb_atom[i]], [EA])
        m = _rowcall(
            functools.partial(_concat_body, 1), E, blk_e,
            [_sc_gather(h, idx_s), _sc_gather(h, idx_t), mnew],
            [Wb_concat[i, :EA], Wb_concat[i, EA:2 * EA], Wb_concat[i, 2 * EA:],
             Wb_res3[i, 0, 0], Wb_res3[i, 0, 1]], [EE])
        Eacc = Eacc + outblock(i + 1, h, m)

    return jax.ops.segment_sum(Eacc[:, 0], batch, num_segments=NG)


# speedup vs baseline: 3.5004x; 1.1234x over previous
import functools

import jax
import jax.numpy as jnp
import numpy as np
from jax import lax
from jax.experimental import pallas as pl
from jax.experimental.pallas import tpu as pltpu
from jax.experimental.pallas import tpu_sc as plsc

_SC_CORES = 2
_SC_SUBCORES = 16
_SC_WORKERS = _SC_CORES * _SC_SUBCORES


def _sc_chunking(bw):
    ch = 0
    for c in range(128, 7, -1):
        if c % 8 == 0 and bw % c == 0:
            ch = c
            break
    nch = bw // ch
    fire = 1
    for k in (8, 5, 4, 2):
        if nch % k == 0:
            fire = k
            break
    return ch, nch, fire


def _sc_gather(table, idx):
    r, d = table.shape
    b = idx.shape[0]
    bw = b // _SC_WORKERS
    ch, nch, fire = _sc_chunking(bw)
    ng = nch // fire
    grp = ch * fire
    idx2 = idx.reshape(_SC_WORKERS, nch, ch)

    mesh = plsc.VectorSubcoreMesh(core_axis_name="c", subcore_axis_name="s")

    @functools.partial(
        pl.kernel,
        out_type=jax.ShapeDtypeStruct((b, d), jnp.float32),
        mesh=mesh,
        scratch_types=[
            pltpu.VMEM((nch, ch), jnp.int32),
            pltpu.VMEM((grp, d), jnp.float32),
            pltpu.SemaphoreType.DMA,
        ],
        compiler_params=pltpu.CompilerParams(use_tc_tiling_on_sc=False),
    )
    def gather_kernel(table_hbm, idx_hbm, out_hbm, idx_v, rows_v, sem):
        wid = lax.axis_index("s") * _SC_CORES + lax.axis_index("c")
        base = wid * bw
        pltpu.sync_copy(idx_hbm.at[wid], idx_v)

        def group(g, _):
            cps = []
            for bb in range(fire):
                cps.append(pltpu.async_copy(
                    table_hbm.at[idx_v.at[g * fire + bb]],
                    rows_v.at[pl.ds(bb * ch, ch)], sem))
            for cp in cps:
                cp.wait()
            pltpu.sync_copy(rows_v, out_hbm.at[pl.ds(base + g * grp, grp)])
            return 0

        lax.fori_loop(0, ng, group, 0)

    return gather_kernel(table, idx2)


def _sc_segsum(rows, idx, n):
    b, d = rows.shape
    bw = b // _SC_WORKERS
    ch, nch, fire = _sc_chunking(bw)
    ng = nch // fire
    grp = ch * fire
    idx2 = idx.reshape(_SC_WORKERS, nch, ch)
    zer = jnp.zeros((n, d), jnp.float32)

    shard = -(-n // _SC_SUBCORES)
    shard += (-shard) % 8
    shares = []
    off = 0
    for s in range(_SC_SUBCORES):
        sz = min(shard, n - off)
        shares.append((off, max(sz, 0)))
        off += sz
    mesh = plsc.VectorSubcoreMesh(core_axis_name="c", subcore_axis_name="s")

    @functools.partial(
        pl.kernel,
        out_type=jax.ShapeDtypeStruct((2 * n, d), jnp.float32),
        mesh=mesh,
        scratch_types=[
            pltpu.VMEM((nch, ch), jnp.int32),
            pltpu.VMEM((grp, d), jnp.float32),
            pltpu.VMEM_SHARED((n, d), jnp.float32),
        ],
        compiler_params=pltpu.CompilerParams(use_tc_tiling_on_sc=False),
    )
    def segsum_kernel(rows_hbm, idx_hbm, zer_hbm, out_hbm, idx_v, rows_v, acc):
        cid = lax.axis_index("c")
        sid = lax.axis_index("s")
        wid = sid * _SC_CORES + cid
        base = wid * bw
        pltpu.sync_copy(idx_hbm.at[wid], idx_v)
        for s in range(_SC_SUBCORES):
            soff, ssz = shares[s]
            if ssz > 0:
                @pl.when(sid == s)
                def _():
                    pltpu.sync_copy(zer_hbm.at[pl.ds(soff, ssz)],
                                    acc.at[pl.ds(soff, ssz)])
        plsc.subcore_barrier()

        def group(g, _):
            pltpu.sync_copy(rows_hbm.at[pl.ds(base + g * grp, grp)], rows_v)
            for bb in range(fire):
                pltpu.sync_copy(rows_v.at[pl.ds(bb * ch, ch)],
                                acc.at[idx_v.at[g * fire + bb]], add=True)
            return 0

        lax.fori_loop(0, ng, group, 0)
        plsc.subcore_barrier()
        for s in range(_SC_SUBCORES):
            soff, ssz = shares[s]
            if ssz > 0:
                @pl.when(sid == s)
                def _():
                    pltpu.sync_copy(acc.at[pl.ds(soff, ssz)],
                                    out_hbm.at[pl.ds(cid * n + soff, ssz)])

    return segsum_kernel(rows, idx2, zer)

INV2 = 1.0 / np.sqrt(2.0)
CUT = 6.0
NS = 7
NGRAPH = 32


def _swish(x):
    return x * jax.nn.sigmoid(x)


def _ln(x):
    mu = jnp.mean(x, axis=-1, keepdims=True)
    v = jnp.mean((x - mu) * (x - mu), axis=-1, keepdims=True)
    return (x - mu) * jax.lax.rsqrt(v + 1e-5)


def _row_spec(blk, w):
    return pl.BlockSpec((blk, w), lambda i: (i, 0))


def _full_spec(shape):
    nd = len(shape)
    return pl.BlockSpec(shape, lambda i: (0,) * nd)


def _rowcall(body, nrows, blk, row_ins, full_ins, out_widths):
    grid = (nrows // blk,)
    in_specs = [_row_spec(blk, a.shape[1]) for a in row_ins]
    in_specs += [_full_spec(a.shape) for a in full_ins]
    out_specs = [_row_spec(blk, w) for w in out_widths]
    out_shape = [jax.ShapeDtypeStruct((nrows, w), jnp.float32) for w in out_widths]
    if len(out_widths) == 1:
        out_specs = out_specs[0]
        out_shape = out_shape[0]
    return pl.pallas_call(
        body,
        grid=grid,
        in_specs=in_specs,
        out_specs=out_specs,
        out_shape=out_shape,
        compiler_params=pltpu.CompilerParams(
            dimension_semantics=("arbitrary",)),
    )(*row_ins, *full_ins)


def _k1_body(nr, d_ref, wr3_ref, wrh_ref, wro_ref, wcbf_ref,
             rbf_ref, r3_ref, rh_ref, ro_ref, renv_ref):
    d = d_ref[...]
    d5 = d * d * d * d * d
    env = 1.0 - 21.0 * d5 + 35.0 * d5 * d - 15.0 * d5 * d * d
    env = jnp.where(d < 1.0, env, 0.0)
    step = 1.0 / (nr - 1)
    offs = jax.lax.broadcasted_iota(
        jnp.int32, (1, nr), 1).astype(jnp.float32) * step
    coeff = -0.5 / (step * step)
    delta = d - offs
    rbf = jnp.exp(coeff * delta * delta) * env
    rbf_ref[...] = rbf
    r3_ref[...] = jnp.dot(rbf, wr3_ref[...], preferred_element_type=jnp.float32)
    rh_ref[...] = jnp.dot(rbf, wrh_ref[...], preferred_element_type=jnp.float32)
    ro_ref[...] = jnp.dot(rbf, wro_ref[...], preferred_element_type=jnp.float32)
    renv_ref[...] = jnp.dot(rbf, wcbf_ref[...], preferred_element_type=jnp.float32)


def _concat_body(res_w, a_ref, b_ref, c_ref, w1_ref, w2_ref, w3_ref, *rest):
    out_ref = rest[-1]
    x = jnp.dot(a_ref[...], w1_ref[...], preferred_element_type=jnp.float32)
    x = x + jnp.dot(b_ref[...], w2_ref[...], preferred_element_type=jnp.float32)
    x = x + jnp.dot(c_ref[...], w3_ref[...], preferred_element_type=jnp.float32)
    x = _swish(x)
    for j in range(res_w):
        wa = rest[2 * j][...]
        wb = rest[2 * j + 1][...]
        t = _swish(jnp.dot(x, wa, preferred_element_type=jnp.float32))
        t = _swish(jnp.dot(t, wb, preferred_element_type=jnp.float32))
        x = (x + t) * INV2
    out_ref[...] = x


def _k3_body(m_ref, r3_ref, wdb_ref, wr3p_ref, wdown_ref, xt_ref):
    xb = _swish(jnp.dot(m_ref[...], wdb_ref[...], preferred_element_type=jnp.float32))
    xb = xb * jnp.dot(r3_ref[...], wr3p_ref[...], preferred_element_type=jnp.float32)
    xt_ref[...] = _swish(jnp.dot(xb, wdown_ref[...], preferred_element_type=jnp.float32))


def _k4_body(cbf_ref, x3g_ref, wcbfp_ref, x3_ref):
    x3_ref[...] = x3g_ref[...] * jnp.dot(
        cbf_ref[...], wcbfp_ref[...], preferred_element_type=jnp.float32)


def _k5_body(m_ref, xe_ref, rh_ref, wbil_ref, wup_ref, wres1_ref, wres2_ref,
             whp_ref, mnew_ref, mscaled_ref):
    xe = _swish(jnp.dot(xe_ref[...], wbil_ref[...], preferred_element_type=jnp.float32))
    xe = _swish(jnp.dot(xe, wup_ref[...], preferred_element_type=jnp.float32))
    x = (m_ref[...] + xe) * INV2
    w1 = wres1_ref[...]
    t = _swish(jnp.dot(x, w1[0], preferred_element_type=jnp.float32))
    t = _swish(jnp.dot(t, w1[1], preferred_element_type=jnp.float32))
    x = (x + t) * INV2
    w2 = wres2_ref[...]
    for j in range(2):
        t = _swish(jnp.dot(x, w2[j, 0], preferred_element_type=jnp.float32))
        t = _swish(jnp.dot(t, w2[j, 1], preferred_element_type=jnp.float32))
        x = (x + t) * INV2
    mnew_ref[...] = x
    mscaled_ref[...] = x * jnp.dot(
        rh_ref[...], whp_ref[...], preferred_element_type=jnp.float32)


def _k6a_body(xa0_ref, xa1_ref, h_ref, watom_ref, hnew_ref):
    x = xa0_ref[...] + xa1_ref[...]
    w = watom_ref[...]
    for l in range(w.shape[0]):
        x = (x + _swish(jnp.dot(x, w[l], preferred_element_type=jnp.float32))) * INV2
    hnew_ref[...] = (h_ref[...] + x) * INV2


def _k6b_body(xa0_ref, xa1_ref, watom_ref, wfin_ref, e_ref):
    x = xa0_ref[...] + xa1_ref[...]
    w = watom_ref[...]
    for l in range(w.shape[0]):
        x = (x + _swish(jnp.dot(x, w[l], preferred_element_type=jnp.float32))) * INV2
    x = _ln(x)
    e_ref[...] = jnp.dot(x, wfin_ref[...], preferred_element_type=jnp.float32)


def _k8_body(m_ref, ro_ref, worbfp_ref, x_ref):
    x_ref[...] = _ln(m_ref[...]) * jnp.dot(
        ro_ref[...], worbfp_ref[...], preferred_element_type=jnp.float32)


def _pick_blk(n, want):
    b = min(want, n)
    while n % b:
        b -= 1
    return b


def kernel(atomic_numbers, pos, edge_index, id3_ba, id3_ca, batch, atom_table,
           W_edge, W_rbf3, W_cbf3, W_rbf_h, W_rbf_out, Wb_db, Wb_rbf3p,
           Wb_down, Wb_cbfp, Wb_bil, Wb_up, Wb_res1, Wb_res2, Wb_res3,
           Wb_rbf_hp, Wb_atom, Wb_concat, Wo_rbfp, Wo_atom, Wo_final):
    N = pos.shape[0]
    E = edge_index.shape[1]
    T = id3_ba.shape[0]
    NR = W_rbf3.shape[0]
    EA = atom_table.shape[1]
    EE = W_edge.shape[1]
    NB = Wb_db.shape[0]
    NG = NGRAPH

    blk_e = _pick_blk(E, 640)
    blk_t = _pick_blk(T, 640)
    blk_n = _pick_blk(N, 1000)

    idx_s = edge_index[0]
    idx_t = edge_index[1]

    vec = pos[idx_t] - pos[idx_s]
    D = jnp.sqrt(jnp.sum(vec * vec, axis=-1) + 1e-12)
    V = vec / D[:, None]
    d = (D / CUT)[:, None]

    wcbf = jnp.transpose(W_cbf3, (1, 0, 2)).reshape(NR, NS * W_cbf3.shape[2])
    rbf, rbf3, rbf_hp, rbf_outp, rbf_env = _rowcall(
        functools.partial(_k1_body, NR), E, blk_e,
        [d], [W_rbf3, W_rbf_h, W_rbf_out, wcbf],
        [NR, W_rbf3.shape[1], W_rbf_h.shape[1], W_rbf_out.shape[1], wcbf.shape[1]])

    Vp = jnp.concatenate([V, jnp.zeros((E, 13), jnp.float32)], axis=1)
    Vba = _sc_gather(Vp, id3_ba)
    Vca = _sc_gather(Vp, id3_ca)
    cosang = jnp.clip(jnp.sum(Vba * Vca, axis=-1), -1.0, 1.0)
    sph_list = [jnp.ones_like(cosang), cosang]
    for l in range(2, NS):
        sph_list.append(
            ((2 * l - 1) * cosang * sph_list[l - 1] - (l - 1) * sph_list[l - 2]) / l)
    sph = jnp.stack(sph_list, axis=1)
    ECBF = W_cbf3.shape[2]
    renvg = _sc_gather(rbf_env, id3_ca)
    cbf_t = jnp.sum(renvg.reshape(T, NS, ECBF) * sph[:, :, None], axis=1)

    h = atom_table[atomic_numbers]

    w1, w2, w3 = W_edge[:EA], W_edge[EA:2 * EA], W_edge[2 * EA:]
    m = _rowcall(functools.partial(_concat_body, 0), E, blk_e,
                 [_sc_gather(h, idx_s), _sc_gather(h, idx_t), rbf],
                 [w1, w2, w3], [EE])

    def outblock(i, h_in, m_in):
        x = _rowcall(_k8_body, E, blk_e, [m_in, rbf_outp], [Wo_rbfp[i]], [EE])
        pp = _sc_segsum(x, idx_t, N)
        return _rowcall(_k6b_body, N, blk_n, [pp[:N], pp[N:]],
                        [Wo_atom[i], Wo_final[i]], [1])

    Eacc = outblock(0, h, m)
    for i in range(NB):
        xt = _rowcall(_k3_body, E, blk_e, [m, rbf3],
                      [Wb_db[i], Wb_rbf3p[i], Wb_down[i]], [Wb_down.shape[2]])
        x3 = _rowcall(_k4_body, T, blk_t, [cbf_t, _sc_gather(xt, id3_ba)],
                      [Wb_cbfp[i]], [Wb_cbfp.shape[2]])
        xe = jax.ops.segment_sum(x3, id3_ca, num_segments=E)
        mnew, mscaled = _rowcall(
            _k5_body, E, blk_e, [m, xe, rbf_hp],
            [Wb_bil[i], Wb_up[i], Wb_res1[i, 0], Wb_res2[i], Wb_rbf_hp[i]],
            [EE, EE])
        pp = _sc_segsum(mscaled, idx_t, N)
        h = _rowcall(_k6a_body, N, blk_n, [pp[:N], pp[N:], h], [Wb_atom[i]], [EA])
        m = _rowcall(
            functools.partial(_concat_body, 1), E, blk_e,
            [_sc_gather(h, idx_s), _sc_gather(h, idx_t), mnew],
            [Wb_concat[i, :EA], Wb_concat[i, EA:2 * EA], Wb_concat[i, 2 * EA:],
             Wb_res3[i, 0, 0], Wb_res3[i, 0, 1]], [EE])
        Eacc = Eacc + outblock(i + 1, h, m)

    return jax.ops.segment_sum(Eacc[:, 0], batch, num_segments=NG)
